# baseline (speedup 1.0000x reference)
import functools
from typing import Any

import jax
import jax.numpy as jnp
from jax.experimental import pallas as pl
from jax.experimental.pallas import tpu as pltpu

EMBED_DIM = 768
FF_DIM = 3072
NUM_EXPERTS = 8
TOP_K = 2
CAPACITY_FACTOR = 2.0
LOAD_BALANCE_WEIGHT = 0.01
ROUTER_Z_WEIGHT = 0.001

BT = 256
BF = 768


def _ffn_body(tok_ref, bexp_ref, nb_ref, x_ref, wg_ref, wu_ref, wo_ref,
              y_ref, xg_ref):
    b = pl.program_id(0)
    j = pl.program_id(1)
    active = b < nb_ref[0]

    @pl.when(active & (j == 0))
    def _gather():
        def gath(i, carry):
            t = tok_ref[b * BT + i]
            xg_ref[i, :] = x_ref[t, :]
            return carry
        jax.lax.fori_loop(0, BT, gath, 0)

    @pl.when(active)
    def _compute():
        xb = xg_ref[:, :]
        wg = wg_ref[0]
        wu = wu_ref[0]
        wo = wo_ref[0]
        dn = (((1,), (1,)), ((), ()))
        g = jax.lax.dot_general(xb, wg, dn, preferred_element_type=jnp.float32)
        u = jax.lax.dot_general(xb, wu, dn, preferred_element_type=jnp.float32)
        h = (g * jax.nn.sigmoid(g)) * u
        yb = jax.lax.dot_general(h, wo, dn, preferred_element_type=jnp.float32)

        @pl.when(j == 0)
        def _():
            y_ref[:, :] = yb

        @pl.when(j > 0)
        def _():
            y_ref[:, :] = y_ref[:, :] + yb


def _grouped_ffn(tok_ids, bexp, nb, xf, wi_gate, wi_up, wo, gmax, nbmax):
    T, D = xf.shape
    nf = FF_DIM // BF
    grid_spec = pltpu.PrefetchScalarGridSpec(
        num_scalar_prefetch=3,
        grid=(nbmax, nf),
        in_specs=[
            pl.BlockSpec((T, D), lambda b, j, tok, bexp, nb: (0, 0)),
            pl.BlockSpec((1, BF, D),
                         lambda b, j, tok, bexp, nb: (bexp[b], j, 0)),
            pl.BlockSpec((1, BF, D),
                         lambda b, j, tok, bexp, nb: (bexp[b], j, 0)),
            pl.BlockSpec((1, D, BF),
                         lambda b, j, tok, bexp, nb: (bexp[b], 0, j)),
        ],
        out_specs=pl.BlockSpec((BT, D), lambda b, j, tok, bexp, nb: (b, 0)),
        scratch_shapes=[pltpu.VMEM((BT, D), jnp.float32)],
    )
    return pl.pallas_call(
        _ffn_body,
        grid_spec=grid_spec,
        out_shape=jax.ShapeDtypeStruct((gmax, D), jnp.float32),
    )(tok_ids, bexp, nb, xf, wi_gate, wi_up, wo)


def kernel(x, gate_w, wi_gate, wi_up, wo):
    B, S, D = x.shape
    T = B * S
    E = NUM_EXPERTS
    cap = max(int(T * TOP_K / E * CAPACITY_FACTOR), TOP_K)
    xf = x.reshape(T, D)

    logits = xf @ gate_w.T
    probs = jax.nn.softmax(logits, axis=-1)
    i0 = jnp.argmax(probs, axis=-1)
    p0 = jnp.max(probs, axis=-1)
    e_ids = jnp.arange(E, dtype=jnp.int32)
    masked = jnp.where(i0[:, None] == e_ids[None, :], -jnp.inf, probs)
    i1 = jnp.argmax(masked, axis=-1)
    p1 = jnp.max(masked, axis=-1)
    s = p0 + p1
    w0 = p0 / s
    w1 = p1 / s

    oh0 = (i0[:, None] == e_ids[None, :]).astype(jnp.int32)
    oh1 = (i1[:, None] == e_ids[None, :]).astype(jnp.int32)
    cum0 = jnp.cumsum(oh0, axis=0)
    cum1 = jnp.cumsum(oh1, axis=0)
    rank0 = jnp.sum(cum0 * oh0, axis=1)
    rank1 = jnp.sum(cum1 * oh1, axis=1)
    kept0 = rank0 <= cap
    kept1 = rank1 <= cap
    cnt0 = jnp.sum((cum0 <= cap) * oh0, axis=0)
    cnt1 = jnp.sum((cum1 <= cap) * oh1, axis=0)
    size = cnt0 + cnt1
    padded = ((size + BT - 1) // BT) * BT
    off = jnp.concatenate([jnp.zeros((1,), jnp.int32),
                           jnp.cumsum(padded)[:-1].astype(jnp.int32)])
    nb = jnp.sum(padded, dtype=jnp.int32) // BT

    gmax = T * TOP_K + E * BT
    nbmax = gmax // BT

    tarange = jnp.arange(T, dtype=jnp.int32)
    row0 = jnp.where(kept0, off[i0] + rank0 - 1, gmax)
    row1 = jnp.where(kept1, off[i1] + cnt0[i1] + rank1 - 1, gmax)
    tok_ids = jnp.zeros((gmax + 1,), jnp.int32)
    tok_ids = tok_ids.at[row0].set(tarange).at[row1].set(tarange)
    tok_ids = tok_ids[:gmax]

    blk_start = off // BT
    barange = jnp.arange(nbmax, dtype=jnp.int32)
    bexp = jnp.searchsorted(blk_start, barange, side='right').astype(jnp.int32) - 1
    last = jnp.maximum(nb - 1, 0)
    bexp = jnp.where(barange < nb, bexp, bexp[last])
    bexp = jnp.clip(bexp, 0, E - 1)

    y = _grouped_ffn(tok_ids, bexp, nb.reshape(1), xf,
                     wi_gate, wi_up, wo, gmax, nbmax)

    r0 = jnp.where(kept0, row0, 0)
    r1 = jnp.where(kept1, row1, 0)
    w0k = jnp.where(kept0, w0, 0.0)
    w1k = jnp.where(kept1, w1, 0.0)
    out = y[r0] * w0k[:, None] + y[r1] * w1k[:, None]
    output = out.reshape(B, S, D)

    f = (oh0 + oh1).sum(axis=0).astype(jnp.float32) / (T * TOP_K)
    P = probs.mean(axis=0)
    load_balance_loss = E * jnp.sum(f * P)
    lse = jax.scipy.special.logsumexp(logits, axis=-1)
    z_loss = jnp.mean(jnp.square(lse))
    aux_loss = (LOAD_BALANCE_WEIGHT * load_balance_loss
                + ROUTER_Z_WEIGHT * z_loss)
    return (output, aux_loss)
